# baseline (speedup 1.0000x reference)
import functools

import jax
import jax.numpy as jnp
import numpy as np
from jax.experimental import pallas as pl
from jax.experimental.pallas import tpu as pltpu
from jax.experimental.pallas import tpu_sc as plsc

NUM_STEPS = 100
KC = 20
HID = 26
L_N = 10000
E_N = 640000
N_LAYERS = 4

NC = 2
NS = 16
NW = NC * NS
EPW = E_N // NW
CH = 80
NCHUNK = EPW // CH
RPT = L_N // NS
D = 32

@functools.cache
def _sc_mesh():
    return plsc.VectorSubcoreMesh(core_axis_name="c", subcore_axis_name="s",
                                  num_cores=NC, num_subcores=NS)


def _schedule_k(num_steps=NUM_STEPS, s=0.01):
    T = num_steps
    tt = np.arange(0, T + 1, dtype=np.float64)
    f_t = np.cos((np.pi / 2.0) * ((tt / T) + s) / (1.0 + s)) ** 2
    alpha_bars = f_t / f_t[0]
    betas = 1.0 - (alpha_bars[1:] / alpha_bars[:-1])
    betas = np.concatenate([np.zeros(1), betas], axis=0)
    betas = np.minimum(betas, 0.999)
    alphas = 1.0 - betas
    return (jnp.asarray(betas, jnp.float32), jnp.asarray(alphas, jnp.float32),
            jnp.asarray(alpha_bars, jnp.float32))


def _so3vec_to_rotation(v):
    theta = jnp.linalg.norm(v, axis=-1, keepdims=True)
    safe = jnp.maximum(theta, 1e-8)
    k = v / safe
    kx, ky, kz = k[..., 0], k[..., 1], k[..., 2]
    zero = jnp.zeros_like(kx)
    K = jnp.stack([jnp.stack([zero, -kz, ky], -1), jnp.stack([kz, zero, -kx], -1),
                   jnp.stack([-ky, kx, zero], -1)], -2)
    st = jnp.sin(theta)[..., None]
    ct = jnp.cos(theta)[..., None]
    I = jnp.eye(3, dtype=v.dtype)
    return I + st * K + (1.0 - ct) * (K @ K)


def _rotation_to_so3vec(R):
    tr = R[..., 0, 0] + R[..., 1, 1] + R[..., 2, 2]
    cos = jnp.clip((tr - 1.0) / 2.0, -1.0 + 1e-6, 1.0 - 1e-6)
    theta = jnp.arccos(cos)
    w = jnp.stack([R[..., 2, 1] - R[..., 1, 2], R[..., 0, 2] - R[..., 2, 0],
                   R[..., 1, 0] - R[..., 0, 1]], -1)
    axis = w / jnp.maximum(2.0 * jnp.sin(theta)[..., None], 1e-8)
    return axis * theta[..., None]


NB = 5
NOUT = NCHUNK // NB


def _gather_body(table_hbm, rows_hbm, cols_hbm, er_hbm, ec_hbm,
                 idx_r, idx_c, buf_r, buf_c, sem_g, sem_w):
    c = jax.lax.axis_index("c")
    s = jax.lax.axis_index("s")
    w = s * NC + c
    base = w * EPW
    pltpu.sync_copy(rows_hbm.at[pl.ds(w * NCHUNK, NCHUNK)], idx_r)
    pltpu.sync_copy(cols_hbm.at[pl.ds(w * NCHUNK, NCHUNK)], idx_c)

    SUP = NB * CH

    def half(j, pp, drain):
        i0 = (2 * j + pp) * NB
        off = base + i0 * CH

        @pl.when(drain)
        def _():
            pltpu.make_async_copy(buf_r.at[pp], er_hbm.at[pl.ds(off, SUP)],
                                  sem_w.at[pp, 0]).wait()
            pltpu.make_async_copy(buf_c.at[pp], ec_hbm.at[pl.ds(off, SUP)],
                                  sem_w.at[pp, 1]).wait()

        ds = []
        for b in range(NB):
            ds.append(pltpu.async_copy(table_hbm.at[idx_r.at[i0 + b]],
                                       buf_r.at[pp, pl.ds(b * CH, CH)], sem_g))
            ds.append(pltpu.async_copy(table_hbm.at[idx_c.at[i0 + b]],
                                       buf_c.at[pp, pl.ds(b * CH, CH)], sem_g))
        for d in ds:
            d.wait()
        pltpu.async_copy(buf_r.at[pp], er_hbm.at[pl.ds(off, SUP)],
                         sem_w.at[pp, 0])
        pltpu.async_copy(buf_c.at[pp], ec_hbm.at[pl.ds(off, SUP)],
                         sem_w.at[pp, 1])

    def outer(j, carry):
        half(j, 0, j > 0)
        half(j, 1, j > 0)
        return carry

    jax.lax.fori_loop(0, NOUT // 2, outer, 0)
    for pp in range(2):
        pltpu.make_async_copy(buf_r.at[pp], er_hbm.at[pl.ds(base, SUP)],
                              sem_w.at[pp, 0]).wait()
        pltpu.make_async_copy(buf_c.at[pp], ec_hbm.at[pl.ds(base, SUP)],
                              sem_w.at[pp, 1]).wait()


def _sc_gather(table, rows2, cols2):
    return pl.kernel(
        _gather_body,
        out_type=[jax.ShapeDtypeStruct((E_N, D), jnp.float32),
                  jax.ShapeDtypeStruct((E_N, D), jnp.float32)],
        mesh=_sc_mesh(),
        compiler_params=pltpu.CompilerParams(use_tc_tiling_on_sc=False),
        scratch_types=[
            pltpu.VMEM((NCHUNK, CH), jnp.int32),
            pltpu.VMEM((NCHUNK, CH), jnp.int32),
            pltpu.VMEM((2, NB * CH, D), jnp.float32),
            pltpu.VMEM((2, NB * CH, D), jnp.float32),
            pltpu.SemaphoreType.DMA,
            pltpu.SemaphoreType.DMA((2, 2)),
        ],
    )(table, rows2, cols2)


def _scatter_body(vals_hbm, rows_hbm, zeros_hbm, out_hbm,
                  idx_v, val_v, zbuf, obuf, acc_sh, sem_v):
    c = jax.lax.axis_index("c")
    s = jax.lax.axis_index("s")
    r0 = s * RPT
    w = c * NS + s
    base = w * EPW
    pltpu.sync_copy(rows_hbm.at[pl.ds(w * NCHUNK, NCHUNK)], idx_v)
    pltpu.sync_copy(zeros_hbm, zbuf)
    pltpu.sync_copy(zbuf, acc_sh.at[pl.ds(r0, RPT)])
    plsc.subcore_barrier()

    def outer(j, carry):
        i0 = j * NB
        ds = []
        for b in range(NB):
            off = base + (i0 + b) * CH
            ds.append(pltpu.async_copy(vals_hbm.at[pl.ds(off, CH)],
                                       val_v.at[b], sem_v))
        for b in range(NB):
            ds[b].wait()
            pltpu.sync_copy(val_v.at[b], acc_sh.at[idx_v.at[i0 + b]], add=True)
        return carry

    jax.lax.fori_loop(0, NOUT, outer, 0)
    plsc.subcore_barrier()
    pltpu.sync_copy(acc_sh.at[pl.ds(r0, RPT)], obuf)
    pltpu.sync_copy(obuf, out_hbm.at[c, pl.ds(r0, RPT)])


def _sc_scatter(vals, rows2, zeros_rpt):
    return pl.kernel(
        _scatter_body,
        out_type=jax.ShapeDtypeStruct((NC, L_N, D), jnp.float32),
        mesh=_sc_mesh(),
        compiler_params=pltpu.CompilerParams(use_tc_tiling_on_sc=False),
        scratch_types=[
            pltpu.VMEM((NCHUNK, CH), jnp.int32),
            pltpu.VMEM((NB, CH, D), jnp.float32),
            pltpu.VMEM((RPT, D), jnp.float32),
            pltpu.VMEM((RPT, D), jnp.float32),
            pltpu.VMEM_SHARED((L_N, D), jnp.float32),
            pltpu.SemaphoreType.DMA,
        ],
    )(vals, rows2, zeros_rpt)


def _embed_body(feat_ref, x_ref, w_ref, b_ref, out_ref):
    h = jnp.dot(feat_ref[...], w_ref[...], preferred_element_type=jnp.float32)
    h = h + b_ref[...]
    pad = jnp.zeros((L_N, D - HID - 3), jnp.float32)
    out_ref[...] = jnp.concatenate([h, x_ref[...], pad], axis=1)


def _tc_embed(in_feat, x0, w, b):
    return pl.pallas_call(
        _embed_body,
        out_shape=jax.ShapeDtypeStruct((L_N, D), jnp.float32),
    )(in_feat, x0, w, b)


BE4 = 800
G = 4


def _bd4(mat, in_off=0, out_off=0):
    ki, ko = mat.shape
    kk, jj = np.meshgrid(np.arange(ki), np.arange(ko), indexing="ij")
    rows = np.concatenate([32 * g + in_off + kk.ravel() for g in range(G)])
    cols = np.concatenate([32 * g + out_off + jj.ravel() for g in range(G)])
    vals = jnp.tile(jnp.asarray(mat, jnp.float32).ravel(), G)
    return jnp.zeros((128, 128), jnp.float32).at[rows, cols].set(vals)


def _row4(vec, off=0):
    n = np.asarray(vec).shape[-1] if isinstance(vec, np.ndarray) else vec.shape[-1]
    cols = np.concatenate([32 * g + off + np.arange(n) for g in range(G)])
    vals = jnp.tile(jnp.asarray(vec, jnp.float32).ravel(), G)
    return jnp.zeros((1, 128), jnp.float32).at[0, cols].set(vals)


_RSUM_NP = np.zeros((128, 128), np.float32)
for _g in range(G):
    _RSUM_NP[32 * _g + HID:32 * _g + HID + 3, 32 * _g:32 * _g + HID + 3] = 1.0
_MASKX_NP = np.zeros((1, 128), np.float32)
_ONE29_NP = np.zeros((1, 128), np.float32)
for _g in range(G):
    _MASKX_NP[0, 32 * _g + HID:32 * _g + HID + 3] = 1.0
    _ONE29_NP[0, 32 * _g + HID + 3] = 1.0


def _edge_weights(p):
    return (
        _bd4(p["edge_w1"][:HID]),
        _bd4(p["edge_w1"][HID:2 * HID]),
        jnp.asarray(_RSUM_NP),
        _row4(p["edge_w1"][2 * HID]),
        _row4(p["edge_b1"]),
        _bd4(p["edge_w2"]),
        _row4(p["edge_b2"]),
        _bd4(jnp.broadcast_to(p["att_w"], (HID, HID))),
        _row4(jnp.broadcast_to(p["att_b"], (HID,))),
        _bd4(p["coord_w1"]),
        _row4(p["coord_b1"]),
        _bd4(jnp.broadcast_to(p["coord_w2"], (HID, 3)), out_off=HID),
        jnp.asarray(_MASKX_NP),
        jnp.asarray(_ONE29_NP),
    )


def _edge_body(er_ref, ec_ref, w1a, w1b, rsum, w1cr, b1r, w2, b2r, awd, abr,
               cw1, cb1r, cws, maskx, one29, out_ref):
    er = er_ref[...]
    ec = ec_ref[...]
    xd = (er - ec) * maskx[...]
    rb = jnp.dot(xd * xd, rsum[...], preferred_element_type=jnp.float32)
    cd = xd / (jnp.sqrt(rb) + 1e-8)
    m = (jnp.dot(er, w1a[...], preferred_element_type=jnp.float32)
         + jnp.dot(ec, w1b[...], preferred_element_type=jnp.float32)
         + rb * w1cr[...] + b1r[...])
    m = m * jax.nn.sigmoid(m)
    m = jnp.dot(m, w2[...], preferred_element_type=jnp.float32) + b2r[...]
    m = m * jax.nn.sigmoid(m)
    att = jax.nn.sigmoid(jnp.dot(m, awd[...], preferred_element_type=jnp.float32)
                         + abr[...])
    m = m * att
    tf = jnp.dot(m, cw1[...], preferred_element_type=jnp.float32) + cb1r[...]
    tf = tf * jax.nn.sigmoid(tf)
    scal = jnp.dot(tf, cws[...], preferred_element_type=jnp.float32)
    out_ref[...] = m + cd * scal + one29[...]


def _tc_edge(er4, ec4, wp):
    nblk = (E_N // G) // BE4
    full = [pl.BlockSpec((BE4, 128), lambda i: (i, 0)),
            pl.BlockSpec((BE4, 128), lambda i: (i, 0))] + [
        pl.BlockSpec(w.shape, lambda i, _n=w.ndim: (0,) * _n) for w in wp]
    return pl.pallas_call(
        _edge_body,
        grid=(nblk,),
        in_specs=full,
        out_specs=pl.BlockSpec((BE4, 128), lambda i: (i, 0)),
        out_shape=jax.ShapeDtypeStruct((E_N // G, 128), jnp.float32),
    )(er4, ec4, *wp)


BN = 1000


def _node_body(tab_ref, p0_ref, p1_ref, nw1a, nw1b, nb1, nw2, nb2, out_ref):
    tab = tab_ref[...]
    agg = p0_ref[...] + p1_ref[...]
    h = tab[:, :HID]
    x = tab[:, HID:HID + 3]
    m_agg = agg[:, :HID]
    tsum = agg[:, HID:HID + 3]
    cnt = agg[:, HID + 3:HID + 4]
    x_new = x + tsum / jnp.clip(cnt, 1.0, None)
    o = (jnp.dot(h, nw1a[...], preferred_element_type=jnp.float32)
         + jnp.dot(m_agg, nw1b[...], preferred_element_type=jnp.float32)
         + nb1[...])
    o = o * jax.nn.sigmoid(o)
    o = jnp.dot(o, nw2[...], preferred_element_type=jnp.float32) + nb2[...]
    h_new = h + o
    pad = jnp.zeros((BN, D - HID - 3), jnp.float32)
    out_ref[...] = jnp.concatenate([h_new, x_new, pad], axis=1)


def _tc_node(tab, p0, p1, wp):
    nspec = pl.BlockSpec((BN, D), lambda i: (i, 0))
    return pl.pallas_call(
        _node_body,
        grid=(L_N // BN,),
        in_specs=[nspec, nspec, nspec] + [
            pl.BlockSpec(w.shape, lambda i, _n=w.ndim: (0,) * _n) for w in wp],
        out_specs=nspec,
        out_shape=jax.ShapeDtypeStruct((L_N, D), jnp.float32),
    )(tab, p0, p1, *wp)


def _final_body(tab_ref, aux_ref, ew, eb, sc_ref, o_rot, o_pos, o_kld):
    tab = tab_ref[...]
    aux = aux_ref[...]
    h = tab[:, :HID]
    x = tab[:, HID:HID + 3]
    pred = jnp.dot(h, ew[...], preferred_element_type=jnp.float32) + eb[...]
    logits = pred[:, :KC]
    mx = jnp.max(logits, axis=1, keepdims=True)
    ex = jnp.exp(logits - mx)
    cden = ex / jnp.sum(ex, axis=1, keepdims=True)
    q = pred[:, KC:KC + 3]
    b_ = q[:, 0:1]
    c_ = q[:, 1:2]
    d_ = q[:, 2:3]
    sq = jnp.sqrt(1.0 + b_ * b_ + c_ * c_ + d_ * d_)
    a_ = 1.0 / sq
    b_ = b_ / sq
    c_ = c_ / sq
    d_ = d_ / sq
    U = [[a_ * a_ + b_ * b_ - c_ * c_ - d_ * d_, 2 * b_ * c_ - 2 * a_ * d_,
          2 * b_ * d_ + 2 * a_ * c_],
         [2 * b_ * c_ + 2 * a_ * d_, a_ * a_ - b_ * b_ + c_ * c_ - d_ * d_,
          2 * c_ * d_ - 2 * a_ * b_],
         [2 * b_ * d_ - 2 * a_ * c_, 2 * c_ * d_ + 2 * a_ * b_,
          a_ * a_ - b_ * b_ - c_ * c_ + d_ * d_]]
    Rn = [[aux[:, 3 * i + j:3 * i + j + 1] for j in range(3)] for i in range(3)]
    R0 = [[aux[:, 9 + 3 * i + j:9 + 3 * i + j + 1] for j in range(3)]
          for i in range(3)]
    Rp = [[Rn[i][0] * U[0][j] + Rn[i][1] * U[1][j] + Rn[i][2] * U[2][j]
           for j in range(3)] for i in range(3)]
    loss_rot = jnp.zeros((BN, 1), jnp.float32)
    for j in range(3):
        dot = Rp[0][j] * R0[0][j] + Rp[1][j] * R0[1][j] + Rp[2][j] * R0[2][j]
        na = jnp.sqrt(Rp[0][j] ** 2 + Rp[1][j] ** 2 + Rp[2][j] ** 2)
        nb = jnp.sqrt(R0[0][j] ** 2 + R0[1][j] ** 2 + R0[2][j] ** 2)
        cos = dot / jnp.maximum(na * nb, 1e-8)
        loss_rot = loss_rot + (1.0 - cos)
    pn = aux[:, 18:21]
    ep = aux[:, 21:24]
    dd = (x - pn) - ep
    pos = jnp.sum(dd * dd, axis=1, keepdims=True)
    alpha = sc_ref[0]
    abp = sc_ref[1]
    c0h = aux[:, 24:24 + KC]
    cth = aux[:, 24 + KC:24 + 2 * KC]
    t1 = alpha * cth + (1.0 - alpha) / KC
    th_true = t1 * (abp * c0h + (1.0 - abp) / KC)
    pt = th_true / (jnp.sum(th_true, axis=1, keepdims=True) + 1e-8)
    th_pred = t1 * (abp * cden + (1.0 - abp) / KC)
    pp = th_pred / (jnp.sum(th_pred, axis=1, keepdims=True) + 1e-8)
    lpp = jnp.log(pp + 1e-8)
    xlx = jnp.where(pt > 0.0, pt * jnp.log(jnp.maximum(pt, 1e-30)), 0.0)
    kld = jnp.sum(xlx - pt * lpp, axis=1, keepdims=True)

    @pl.when(pl.program_id(0) == 0)
    def _init():
        o_rot[...] = jnp.zeros((1, 1), jnp.float32)
        o_pos[...] = jnp.zeros((1, 1), jnp.float32)
        o_kld[...] = jnp.zeros((1, 1), jnp.float32)

    o_rot[...] += jnp.sum(loss_rot, keepdims=True)
    o_pos[...] += jnp.sum(pos, keepdims=True)
    o_kld[...] += jnp.sum(kld, keepdims=True)


def _tc_final(tab, aux, ew, eb, scal):
    ospec = pl.BlockSpec((1, 1), lambda i: (0, 0))
    return pl.pallas_call(
        _final_body,
        grid=(L_N // BN,),
        in_specs=[
            pl.BlockSpec((BN, D), lambda i: (i, 0)),
            pl.BlockSpec((BN, 64), lambda i: (i, 0)),
            pl.BlockSpec((HID, 23), lambda i: (0, 0)),
            pl.BlockSpec((1, 23), lambda i: (0, 0)),
            pl.BlockSpec(memory_space=pltpu.SMEM),
        ],
        out_specs=[ospec, ospec, ospec],
        out_shape=[jax.ShapeDtypeStruct((1, 1), jnp.float32)] * 3,
    )(tab, aux, ew, eb, scal)


def kernel(p_0, c_0, v_0, e, t, params):
    betas, alphas, alpha_bars = _schedule_k()
    nkey = jax.random.key(1234)
    k1, k2, k3, k4 = jax.random.split(nkey, 4)
    N, L = p_0.shape[0], p_0.shape[1]
    rows, cols = e[0], e[1]
    p_0n = p_0 / 10.0
    R_0 = _so3vec_to_rotation(v_0)
    ab_t = alpha_bars[t]
    c0 = jnp.sqrt(ab_t)[:, None, None]
    c1 = jnp.sqrt(1.0 - ab_t)[:, None, None]
    e_scaled = c1 * jax.random.normal(k1, v_0.shape, jnp.float32)
    E_scaled = _so3vec_to_rotation(e_scaled)
    R0_scaled = _so3vec_to_rotation(c0 * v_0)
    R_noisy = E_scaled @ R0_scaled
    v_noisy = _rotation_to_so3vec(R_noisy)
    eps_p = jax.random.normal(k2, p_0.shape, jnp.float32)
    p_noisy = c0 * p_0n + c1 * eps_p
    s_0 = jax.random.categorical(
        k3, jnp.log(jnp.clip(jax.lax.stop_gradient(c_0), 1e-8, None)), axis=-1)
    c0h = jax.nn.one_hot(s_0, KC, dtype=jnp.float32)
    c_noisy = ab_t[:, None, None] * c0h + (1.0 - ab_t[:, None, None]) / KC
    s_noisy = jax.random.categorical(
        k4, jnp.log(jnp.clip(jax.lax.stop_gradient(c_noisy), 1e-8, None)), axis=-1)
    beta = betas[t]
    t_embed = jnp.stack([beta, jnp.sin(beta), jnp.cos(beta)], axis=-1)
    t_embed = jnp.broadcast_to(t_embed[0][None, :], (L, 3))
    in_feat = jnp.concatenate([c_noisy[0], v_noisy[0], t_embed], axis=1)

    tab = _tc_embed(in_feat, p_noisy[0], params["emb_in_w"],
                    params["emb_in_b"].reshape(1, HID))
    zeros_rpt = jnp.zeros((RPT, D), jnp.float32)
    rows2 = rows.reshape(E_N // CH, CH)
    cols2 = cols.reshape(E_N // CH, CH)
    for p in params["layers"]:
        ewp = _edge_weights(p)
        nwp = (p["node_w1"][:HID], p["node_w1"][HID:2 * HID],
               p["node_b1"].reshape(1, HID), p["node_w2"],
               p["node_b2"].reshape(1, HID))
        er, ec = _sc_gather(tab, rows2, cols2)
        vals4 = _tc_edge(er.reshape(E_N // G, 128), ec.reshape(E_N // G, 128),
                         ewp)
        part = _sc_scatter(vals4.reshape(E_N, D), rows2, zeros_rpt)
        tab = _tc_node(tab, part[0], part[1], nwp)

    ct_h = jax.nn.one_hot(s_noisy, KC, dtype=jnp.float32)
    aux = jnp.concatenate([
        R_noisy[0].reshape(L, 9), R_0[0].reshape(L, 9),
        p_noisy[0], eps_p[0], c0h[0], ct_h[0]], axis=1)
    scal = jnp.stack([alphas[t][0],
                      alpha_bars[(t - 1) % alpha_bars.shape[0]][0]])
    s_rot, s_pos, s_kld = _tc_final(tab, aux, params["emb_out_w"],
                                    params["emb_out_b"].reshape(1, 23), scal)
    denom = float(N * L) + 1e-8
    return jnp.stack([s_rot[0, 0] / denom, s_pos[0, 0] / denom,
                      s_kld[0, 0] / denom])


# speedup vs baseline: 1.0960x; 1.0960x over previous
import functools

import jax
import jax.numpy as jnp
import numpy as np
from jax.experimental import pallas as pl
from jax.experimental.pallas import tpu as pltpu
from jax.experimental.pallas import tpu_sc as plsc

NUM_STEPS = 100
KC = 20
HID = 26
L_N = 10000
E_N = 640000
N_LAYERS = 4

NC = 2
NS = 16
NW = NC * NS
EPW = E_N // NW
CH = 80
NCHUNK = EPW // CH
RPT = L_N // NS
D = 32

@functools.cache
def _sc_mesh():
    return plsc.VectorSubcoreMesh(core_axis_name="c", subcore_axis_name="s",
                                  num_cores=NC, num_subcores=NS)


def _schedule_k(num_steps=NUM_STEPS, s=0.01):
    T = num_steps
    tt = np.arange(0, T + 1, dtype=np.float64)
    f_t = np.cos((np.pi / 2.0) * ((tt / T) + s) / (1.0 + s)) ** 2
    alpha_bars = f_t / f_t[0]
    betas = 1.0 - (alpha_bars[1:] / alpha_bars[:-1])
    betas = np.concatenate([np.zeros(1), betas], axis=0)
    betas = np.minimum(betas, 0.999)
    alphas = 1.0 - betas
    return (jnp.asarray(betas, jnp.float32), jnp.asarray(alphas, jnp.float32),
            jnp.asarray(alpha_bars, jnp.float32))


def _so3vec_to_rotation(v):
    theta = jnp.linalg.norm(v, axis=-1, keepdims=True)
    safe = jnp.maximum(theta, 1e-8)
    k = v / safe
    kx, ky, kz = k[..., 0], k[..., 1], k[..., 2]
    zero = jnp.zeros_like(kx)
    K = jnp.stack([jnp.stack([zero, -kz, ky], -1), jnp.stack([kz, zero, -kx], -1),
                   jnp.stack([-ky, kx, zero], -1)], -2)
    st = jnp.sin(theta)[..., None]
    ct = jnp.cos(theta)[..., None]
    I = jnp.eye(3, dtype=v.dtype)
    return I + st * K + (1.0 - ct) * (K @ K)


def _rotation_to_so3vec(R):
    tr = R[..., 0, 0] + R[..., 1, 1] + R[..., 2, 2]
    cos = jnp.clip((tr - 1.0) / 2.0, -1.0 + 1e-6, 1.0 - 1e-6)
    theta = jnp.arccos(cos)
    w = jnp.stack([R[..., 2, 1] - R[..., 1, 2], R[..., 0, 2] - R[..., 2, 0],
                   R[..., 1, 0] - R[..., 0, 1]], -1)
    axis = w / jnp.maximum(2.0 * jnp.sin(theta)[..., None], 1e-8)
    return axis * theta[..., None]


NB = 5
NOUT = NCHUNK // NB


def _gather_body(table_hbm, rows_hbm, cols_hbm, er_hbm, ec_hbm,
                 idx_r, idx_c, buf_r, buf_c, sem_g, sem_w):
    c = jax.lax.axis_index("c")
    s = jax.lax.axis_index("s")
    w = s * NC + c
    base = w * EPW
    pltpu.sync_copy(rows_hbm.at[pl.ds(w * NCHUNK, NCHUNK)], idx_r)
    pltpu.sync_copy(cols_hbm.at[pl.ds(w * NCHUNK, NCHUNK)], idx_c)

    SUP = NB * CH

    def half(j, pp, drain):
        i0 = (2 * j + pp) * NB
        off = base + i0 * CH

        @pl.when(drain)
        def _():
            pltpu.make_async_copy(buf_r.at[pp], er_hbm.at[pl.ds(off, SUP)],
                                  sem_w.at[pp, 0]).wait()
            pltpu.make_async_copy(buf_c.at[pp], ec_hbm.at[pl.ds(off, SUP)],
                                  sem_w.at[pp, 1]).wait()

        ds = []
        for b in range(NB):
            ds.append(pltpu.async_copy(table_hbm.at[idx_r.at[i0 + b]],
                                       buf_r.at[pp, pl.ds(b * CH, CH)], sem_g))
            ds.append(pltpu.async_copy(table_hbm.at[idx_c.at[i0 + b]],
                                       buf_c.at[pp, pl.ds(b * CH, CH)], sem_g))
        for d in ds:
            d.wait()
        pltpu.async_copy(buf_r.at[pp], er_hbm.at[pl.ds(off, SUP)],
                         sem_w.at[pp, 0])
        pltpu.async_copy(buf_c.at[pp], ec_hbm.at[pl.ds(off, SUP)],
                         sem_w.at[pp, 1])

    def outer(j, carry):
        half(j, 0, j > 0)
        half(j, 1, j > 0)
        return carry

    jax.lax.fori_loop(0, NOUT // 2, outer, 0)
    for pp in range(2):
        pltpu.make_async_copy(buf_r.at[pp], er_hbm.at[pl.ds(base, SUP)],
                              sem_w.at[pp, 0]).wait()
        pltpu.make_async_copy(buf_c.at[pp], ec_hbm.at[pl.ds(base, SUP)],
                              sem_w.at[pp, 1]).wait()


def _sc_gather(table, rows2, cols2):
    return pl.kernel(
        _gather_body,
        out_type=[jax.ShapeDtypeStruct((E_N, D), jnp.float32),
                  jax.ShapeDtypeStruct((E_N, D), jnp.float32)],
        mesh=_sc_mesh(),
        compiler_params=pltpu.CompilerParams(use_tc_tiling_on_sc=False),
        scratch_types=[
            pltpu.VMEM((NCHUNK, CH), jnp.int32),
            pltpu.VMEM((NCHUNK, CH), jnp.int32),
            pltpu.VMEM((2, NB * CH, D), jnp.float32),
            pltpu.VMEM((2, NB * CH, D), jnp.float32),
            pltpu.SemaphoreType.DMA,
            pltpu.SemaphoreType.DMA((2, 2)),
        ],
    )(table, rows2, cols2)


def _scatter_body(vals_hbm, rows_hbm, zeros_hbm, out_hbm,
                  idx_v, val_v, zbuf, obuf, acc_sh, sem_v):
    c = jax.lax.axis_index("c")
    s = jax.lax.axis_index("s")
    r0 = s * RPT
    w = c * NS + s
    base = w * EPW
    pltpu.sync_copy(rows_hbm.at[pl.ds(w * NCHUNK, NCHUNK)], idx_v)
    pltpu.sync_copy(zeros_hbm, zbuf)
    pltpu.sync_copy(zbuf, acc_sh.at[pl.ds(r0, RPT)])
    plsc.subcore_barrier()

    def outer(j, carry):
        i0 = j * NB
        ds = []
        for b in range(NB):
            off = base + (i0 + b) * CH
            ds.append(pltpu.async_copy(vals_hbm.at[pl.ds(off, CH)],
                                       val_v.at[b], sem_v))
        for b in range(NB):
            ds[b].wait()
            pltpu.sync_copy(val_v.at[b], acc_sh.at[idx_v.at[i0 + b]], add=True)
        return carry

    jax.lax.fori_loop(0, NOUT, outer, 0)
    plsc.subcore_barrier()
    pltpu.sync_copy(acc_sh.at[pl.ds(r0, RPT)], obuf)
    pltpu.sync_copy(obuf, out_hbm.at[c, pl.ds(r0, RPT)])


def _sc_scatter(vals, rows2, zeros_rpt):
    return pl.kernel(
        _scatter_body,
        out_type=jax.ShapeDtypeStruct((NC, L_N, D), jnp.float32),
        mesh=_sc_mesh(),
        compiler_params=pltpu.CompilerParams(use_tc_tiling_on_sc=False),
        scratch_types=[
            pltpu.VMEM((NCHUNK, CH), jnp.int32),
            pltpu.VMEM((NB, CH, D), jnp.float32),
            pltpu.VMEM((RPT, D), jnp.float32),
            pltpu.VMEM((RPT, D), jnp.float32),
            pltpu.VMEM_SHARED((L_N, D), jnp.float32),
            pltpu.SemaphoreType.DMA,
        ],
    )(vals, rows2, zeros_rpt)


def _embed_body(feat_ref, x_ref, w_ref, b_ref, out_ref):
    h = jnp.dot(feat_ref[...], w_ref[...], preferred_element_type=jnp.float32)
    h = h + b_ref[...]
    pad = jnp.zeros((L_N, D - HID - 3), jnp.float32)
    out_ref[...] = jnp.concatenate([h, x_ref[...], pad], axis=1)


def _tc_embed(in_feat, x0, w, b):
    return pl.pallas_call(
        _embed_body,
        out_shape=jax.ShapeDtypeStruct((L_N, D), jnp.float32),
    )(in_feat, x0, w, b)


BE4 = 1600
G = 4


def _bd4(mat, in_off=0, out_off=0):
    ki, ko = mat.shape
    kk, jj = np.meshgrid(np.arange(ki), np.arange(ko), indexing="ij")
    rows = np.concatenate([32 * g + in_off + kk.ravel() for g in range(G)])
    cols = np.concatenate([32 * g + out_off + jj.ravel() for g in range(G)])
    vals = jnp.tile(jnp.asarray(mat, jnp.float32).ravel(), G)
    return jnp.zeros((128, 128), jnp.float32).at[rows, cols].set(vals)


def _row4(vec, off=0):
    n = np.asarray(vec).shape[-1] if isinstance(vec, np.ndarray) else vec.shape[-1]
    cols = np.concatenate([32 * g + off + np.arange(n) for g in range(G)])
    vals = jnp.tile(jnp.asarray(vec, jnp.float32).ravel(), G)
    return jnp.zeros((1, 128), jnp.float32).at[0, cols].set(vals)


_RSUM_NP = np.zeros((128, 128), np.float32)
for _g in range(G):
    _RSUM_NP[32 * _g + HID:32 * _g + HID + 3, 32 * _g:32 * _g + HID + 3] = 1.0
_MASKX_NP = np.zeros((1, 128), np.float32)
_ONE29_NP = np.zeros((1, 128), np.float32)
for _g in range(G):
    _MASKX_NP[0, 32 * _g + HID:32 * _g + HID + 3] = 1.0
    _ONE29_NP[0, 32 * _g + HID + 3] = 1.0


def _edge_weights(p):
    return (
        _bd4(p["edge_w1"][:HID]),
        _bd4(p["edge_w1"][HID:2 * HID]),
        jnp.asarray(_RSUM_NP),
        _row4(p["edge_w1"][2 * HID]),
        _row4(p["edge_b1"]),
        _bd4(p["edge_w2"]),
        _row4(p["edge_b2"]),
        _bd4(jnp.broadcast_to(p["att_w"], (HID, HID))),
        _row4(jnp.broadcast_to(p["att_b"], (HID,))),
        _bd4(p["coord_w1"]),
        _row4(p["coord_b1"]),
        _bd4(jnp.broadcast_to(p["coord_w2"], (HID, 3)), out_off=HID),
        jnp.asarray(_MASKX_NP),
        jnp.asarray(_ONE29_NP),
    )


def _edge_body(er_ref, ec_ref, w1a, w1b, rsum, w1cr, b1r, w2, b2r, awd, abr,
               cw1, cb1r, cws, maskx, one29, out_ref):
    er = er_ref[...]
    ec = ec_ref[...]
    xd = (er - ec) * maskx[...]
    rb = jnp.dot(xd * xd, rsum[...], preferred_element_type=jnp.float32)
    cd = xd / (jnp.sqrt(rb) + 1e-8)
    m = (jnp.dot(er, w1a[...], preferred_element_type=jnp.float32)
         + jnp.dot(ec, w1b[...], preferred_element_type=jnp.float32)
         + rb * w1cr[...] + b1r[...])
    m = m * jax.nn.sigmoid(m)
    m = jnp.dot(m, w2[...], preferred_element_type=jnp.float32) + b2r[...]
    m = m * jax.nn.sigmoid(m)
    att = jax.nn.sigmoid(jnp.dot(m, awd[...], preferred_element_type=jnp.float32)
                         + abr[...])
    m = m * att
    tf = jnp.dot(m, cw1[...], preferred_element_type=jnp.float32) + cb1r[...]
    tf = tf * jax.nn.sigmoid(tf)
    scal = jnp.dot(tf, cws[...], preferred_element_type=jnp.float32)
    out_ref[...] = m + cd * scal + one29[...]


def _tc_edge(er4, ec4, wp):
    nblk = (E_N // G) // BE4
    full = [pl.BlockSpec((BE4, 128), lambda i: (i, 0)),
            pl.BlockSpec((BE4, 128), lambda i: (i, 0))] + [
        pl.BlockSpec(w.shape, lambda i, _n=w.ndim: (0,) * _n) for w in wp]
    return pl.pallas_call(
        _edge_body,
        grid=(nblk,),
        in_specs=full,
        out_specs=pl.BlockSpec((BE4, 128), lambda i: (i, 0)),
        out_shape=jax.ShapeDtypeStruct((E_N // G, 128), jnp.float32),
    )(er4, ec4, *wp)


BN = 1000


def _node_body(tab_ref, p0_ref, p1_ref, nw1a, nw1b, nb1, nw2, nb2, out_ref):
    tab = tab_ref[...]
    agg = p0_ref[...] + p1_ref[...]
    h = tab[:, :HID]
    x = tab[:, HID:HID + 3]
    m_agg = agg[:, :HID]
    tsum = agg[:, HID:HID + 3]
    cnt = agg[:, HID + 3:HID + 4]
    x_new = x + tsum / jnp.clip(cnt, 1.0, None)
    o = (jnp.dot(h, nw1a[...], preferred_element_type=jnp.float32)
         + jnp.dot(m_agg, nw1b[...], preferred_element_type=jnp.float32)
         + nb1[...])
    o = o * jax.nn.sigmoid(o)
    o = jnp.dot(o, nw2[...], preferred_element_type=jnp.float32) + nb2[...]
    h_new = h + o
    pad = jnp.zeros((BN, D - HID - 3), jnp.float32)
    out_ref[...] = jnp.concatenate([h_new, x_new, pad], axis=1)


def _tc_node(tab, p0, p1, wp):
    nspec = pl.BlockSpec((BN, D), lambda i: (i, 0))
    return pl.pallas_call(
        _node_body,
        grid=(L_N // BN,),
        in_specs=[nspec, nspec, nspec] + [
            pl.BlockSpec(w.shape, lambda i, _n=w.ndim: (0,) * _n) for w in wp],
        out_specs=nspec,
        out_shape=jax.ShapeDtypeStruct((L_N, D), jnp.float32),
    )(tab, p0, p1, *wp)


def _final_body(tab_ref, aux_ref, ew, eb, sc_ref, o_rot, o_pos, o_kld):
    tab = tab_ref[...]
    aux = aux_ref[...]
    h = tab[:, :HID]
    x = tab[:, HID:HID + 3]
    pred = jnp.dot(h, ew[...], preferred_element_type=jnp.float32) + eb[...]
    logits = pred[:, :KC]
    mx = jnp.max(logits, axis=1, keepdims=True)
    ex = jnp.exp(logits - mx)
    cden = ex / jnp.sum(ex, axis=1, keepdims=True)
    q = pred[:, KC:KC + 3]
    b_ = q[:, 0:1]
    c_ = q[:, 1:2]
    d_ = q[:, 2:3]
    sq = jnp.sqrt(1.0 + b_ * b_ + c_ * c_ + d_ * d_)
    a_ = 1.0 / sq
    b_ = b_ / sq
    c_ = c_ / sq
    d_ = d_ / sq
    U = [[a_ * a_ + b_ * b_ - c_ * c_ - d_ * d_, 2 * b_ * c_ - 2 * a_ * d_,
          2 * b_ * d_ + 2 * a_ * c_],
         [2 * b_ * c_ + 2 * a_ * d_, a_ * a_ - b_ * b_ + c_ * c_ - d_ * d_,
          2 * c_ * d_ - 2 * a_ * b_],
         [2 * b_ * d_ - 2 * a_ * c_, 2 * c_ * d_ + 2 * a_ * b_,
          a_ * a_ - b_ * b_ - c_ * c_ + d_ * d_]]
    Rn = [[aux[:, 3 * i + j:3 * i + j + 1] for j in range(3)] for i in range(3)]
    R0 = [[aux[:, 9 + 3 * i + j:9 + 3 * i + j + 1] for j in range(3)]
          for i in range(3)]
    Rp = [[Rn[i][0] * U[0][j] + Rn[i][1] * U[1][j] + Rn[i][2] * U[2][j]
           for j in range(3)] for i in range(3)]
    loss_rot = jnp.zeros((BN, 1), jnp.float32)
    for j in range(3):
        dot = Rp[0][j] * R0[0][j] + Rp[1][j] * R0[1][j] + Rp[2][j] * R0[2][j]
        na = jnp.sqrt(Rp[0][j] ** 2 + Rp[1][j] ** 2 + Rp[2][j] ** 2)
        nb = jnp.sqrt(R0[0][j] ** 2 + R0[1][j] ** 2 + R0[2][j] ** 2)
        cos = dot / jnp.maximum(na * nb, 1e-8)
        loss_rot = loss_rot + (1.0 - cos)
    pn = aux[:, 18:21]
    ep = aux[:, 21:24]
    dd = (x - pn) - ep
    pos = jnp.sum(dd * dd, axis=1, keepdims=True)
    alpha = sc_ref[0]
    abp = sc_ref[1]
    c0h = aux[:, 24:24 + KC]
    cth = aux[:, 24 + KC:24 + 2 * KC]
    t1 = alpha * cth + (1.0 - alpha) / KC
    th_true = t1 * (abp * c0h + (1.0 - abp) / KC)
    pt = th_true / (jnp.sum(th_true, axis=1, keepdims=True) + 1e-8)
    th_pred = t1 * (abp * cden + (1.0 - abp) / KC)
    pp = th_pred / (jnp.sum(th_pred, axis=1, keepdims=True) + 1e-8)
    lpp = jnp.log(pp + 1e-8)
    xlx = jnp.where(pt > 0.0, pt * jnp.log(jnp.maximum(pt, 1e-30)), 0.0)
    kld = jnp.sum(xlx - pt * lpp, axis=1, keepdims=True)

    @pl.when(pl.program_id(0) == 0)
    def _init():
        o_rot[...] = jnp.zeros((1, 1), jnp.float32)
        o_pos[...] = jnp.zeros((1, 1), jnp.float32)
        o_kld[...] = jnp.zeros((1, 1), jnp.float32)

    o_rot[...] += jnp.sum(loss_rot, keepdims=True)
    o_pos[...] += jnp.sum(pos, keepdims=True)
    o_kld[...] += jnp.sum(kld, keepdims=True)


def _tc_final(tab, aux, ew, eb, scal):
    ospec = pl.BlockSpec((1, 1), lambda i: (0, 0))
    return pl.pallas_call(
        _final_body,
        grid=(L_N // BN,),
        in_specs=[
            pl.BlockSpec((BN, D), lambda i: (i, 0)),
            pl.BlockSpec((BN, 64), lambda i: (i, 0)),
            pl.BlockSpec((HID, 23), lambda i: (0, 0)),
            pl.BlockSpec((1, 23), lambda i: (0, 0)),
            pl.BlockSpec(memory_space=pltpu.SMEM),
        ],
        out_specs=[ospec, ospec, ospec],
        out_shape=[jax.ShapeDtypeStruct((1, 1), jnp.float32)] * 3,
    )(tab, aux, ew, eb, scal)


def kernel(p_0, c_0, v_0, e, t, params):
    betas, alphas, alpha_bars = _schedule_k()
    nkey = jax.random.key(1234)
    k1, k2, k3, k4 = jax.random.split(nkey, 4)
    N, L = p_0.shape[0], p_0.shape[1]
    rows, cols = e[0], e[1]
    p_0n = p_0 / 10.0
    R_0 = _so3vec_to_rotation(v_0)
    ab_t = alpha_bars[t]
    c0 = jnp.sqrt(ab_t)[:, None, None]
    c1 = jnp.sqrt(1.0 - ab_t)[:, None, None]
    e_scaled = c1 * jax.random.normal(k1, v_0.shape, jnp.float32)
    E_scaled = _so3vec_to_rotation(e_scaled)
    R0_scaled = _so3vec_to_rotation(c0 * v_0)
    R_noisy = E_scaled @ R0_scaled
    v_noisy = _rotation_to_so3vec(R_noisy)
    eps_p = jax.random.normal(k2, p_0.shape, jnp.float32)
    p_noisy = c0 * p_0n + c1 * eps_p
    s_0 = jax.random.categorical(
        k3, jnp.log(jnp.clip(jax.lax.stop_gradient(c_0), 1e-8, None)), axis=-1)
    c0h = jax.nn.one_hot(s_0, KC, dtype=jnp.float32)
    c_noisy = ab_t[:, None, None] * c0h + (1.0 - ab_t[:, None, None]) / KC
    s_noisy = jax.random.categorical(
        k4, jnp.log(jnp.clip(jax.lax.stop_gradient(c_noisy), 1e-8, None)), axis=-1)
    beta = betas[t]
    t_embed = jnp.stack([beta, jnp.sin(beta), jnp.cos(beta)], axis=-1)
    t_embed = jnp.broadcast_to(t_embed[0][None, :], (L, 3))
    in_feat = jnp.concatenate([c_noisy[0], v_noisy[0], t_embed], axis=1)

    tab = _tc_embed(in_feat, p_noisy[0], params["emb_in_w"],
                    params["emb_in_b"].reshape(1, HID))
    zeros_rpt = jnp.zeros((RPT, D), jnp.float32)
    rows2 = rows.reshape(E_N // CH, CH)
    cols2 = cols.reshape(E_N // CH, CH)
    for p in params["layers"]:
        ewp = _edge_weights(p)
        nwp = (p["node_w1"][:HID], p["node_w1"][HID:2 * HID],
               p["node_b1"].reshape(1, HID), p["node_w2"],
               p["node_b2"].reshape(1, HID))
        er, ec = _sc_gather(tab, rows2, cols2)
        vals4 = _tc_edge(er.reshape(E_N // G, 128), ec.reshape(E_N // G, 128),
                         ewp)
        part = _sc_scatter(vals4.reshape(E_N, D), rows2, zeros_rpt)
        tab = _tc_node(tab, part[0], part[1], nwp)

    ct_h = jax.nn.one_hot(s_noisy, KC, dtype=jnp.float32)
    aux = jnp.concatenate([
        R_noisy[0].reshape(L, 9), R_0[0].reshape(L, 9),
        p_noisy[0], eps_p[0], c0h[0], ct_h[0]], axis=1)
    scal = jnp.stack([alphas[t][0],
                      alpha_bars[(t - 1) % alpha_bars.shape[0]][0]])
    s_rot, s_pos, s_kld = _tc_final(tab, aux, params["emb_out_w"],
                                    params["emb_out_b"].reshape(1, 23), scal)
    denom = float(N * L) + 1e-8
    return jnp.stack([s_rot[0, 0] / denom, s_pos[0, 0] / denom,
                      s_kld[0, 0] / denom])


# speedup vs baseline: 1.1241x; 1.0257x over previous
import functools

import jax
import jax.numpy as jnp
import numpy as np
from jax.experimental import pallas as pl
from jax.experimental.pallas import tpu as pltpu
from jax.experimental.pallas import tpu_sc as plsc

NUM_STEPS = 100
KC = 20
HID = 26
L_N = 10000
E_N = 640000
N_LAYERS = 4

NC = 2
NS = 16
NW = NC * NS
EPW = E_N // NW
CH = 80
NCHUNK = EPW // CH
RPT = L_N // NS
D = 32

@functools.cache
def _sc_mesh():
    return plsc.VectorSubcoreMesh(core_axis_name="c", subcore_axis_name="s",
                                  num_cores=NC, num_subcores=NS)


def _schedule_k(num_steps=NUM_STEPS, s=0.01):
    T = num_steps
    tt = np.arange(0, T + 1, dtype=np.float64)
    f_t = np.cos((np.pi / 2.0) * ((tt / T) + s) / (1.0 + s)) ** 2
    alpha_bars = f_t / f_t[0]
    betas = 1.0 - (alpha_bars[1:] / alpha_bars[:-1])
    betas = np.concatenate([np.zeros(1), betas], axis=0)
    betas = np.minimum(betas, 0.999)
    alphas = 1.0 - betas
    return (jnp.asarray(betas, jnp.float32), jnp.asarray(alphas, jnp.float32),
            jnp.asarray(alpha_bars, jnp.float32))


def _so3vec_to_rotation(v):
    theta = jnp.linalg.norm(v, axis=-1, keepdims=True)
    safe = jnp.maximum(theta, 1e-8)
    k = v / safe
    kx, ky, kz = k[..., 0], k[..., 1], k[..., 2]
    zero = jnp.zeros_like(kx)
    K = jnp.stack([jnp.stack([zero, -kz, ky], -1), jnp.stack([kz, zero, -kx], -1),
                   jnp.stack([-ky, kx, zero], -1)], -2)
    st = jnp.sin(theta)[..., None]
    ct = jnp.cos(theta)[..., None]
    I = jnp.eye(3, dtype=v.dtype)
    return I + st * K + (1.0 - ct) * (K @ K)


def _rotation_to_so3vec(R):
    tr = R[..., 0, 0] + R[..., 1, 1] + R[..., 2, 2]
    cos = jnp.clip((tr - 1.0) / 2.0, -1.0 + 1e-6, 1.0 - 1e-6)
    theta = jnp.arccos(cos)
    w = jnp.stack([R[..., 2, 1] - R[..., 1, 2], R[..., 0, 2] - R[..., 2, 0],
                   R[..., 1, 0] - R[..., 0, 1]], -1)
    axis = w / jnp.maximum(2.0 * jnp.sin(theta)[..., None], 1e-8)
    return axis * theta[..., None]


NB = 5
NOUT = NCHUNK // NB


def _gather_body(table_hbm, rows_hbm, cols_hbm, er_hbm, ec_hbm,
                 idx_r, idx_c, buf_r, buf_c, sem_g, sem_w):
    c = jax.lax.axis_index("c")
    s = jax.lax.axis_index("s")
    w = s * NC + c
    base = w * EPW
    pltpu.sync_copy(rows_hbm.at[pl.ds(w * NCHUNK, NCHUNK)], idx_r)
    pltpu.sync_copy(cols_hbm.at[pl.ds(w * NCHUNK, NCHUNK)], idx_c)

    SUP = NB * CH

    def half(j, pp, drain):
        i0 = (2 * j + pp) * NB
        off = base + i0 * CH

        @pl.when(drain)
        def _():
            pltpu.make_async_copy(buf_r.at[pp], er_hbm.at[pl.ds(off, SUP)],
                                  sem_w.at[pp, 0]).wait()
            pltpu.make_async_copy(buf_c.at[pp], ec_hbm.at[pl.ds(off, SUP)],
                                  sem_w.at[pp, 1]).wait()

        ds = []
        for b in range(NB):
            ds.append(pltpu.async_copy(table_hbm.at[idx_r.at[i0 + b]],
                                       buf_r.at[pp, pl.ds(b * CH, CH)], sem_g))
            ds.append(pltpu.async_copy(table_hbm.at[idx_c.at[i0 + b]],
                                       buf_c.at[pp, pl.ds(b * CH, CH)], sem_g))
        for d in ds:
            d.wait()
        pltpu.async_copy(buf_r.at[pp], er_hbm.at[pl.ds(off, SUP)],
                         sem_w.at[pp, 0])
        pltpu.async_copy(buf_c.at[pp], ec_hbm.at[pl.ds(off, SUP)],
                         sem_w.at[pp, 1])

    def outer(j, carry):
        half(j, 0, j > 0)
        half(j, 1, j > 0)
        return carry

    jax.lax.fori_loop(0, NOUT // 2, outer, 0)
    for pp in range(2):
        pltpu.make_async_copy(buf_r.at[pp], er_hbm.at[pl.ds(base, SUP)],
                              sem_w.at[pp, 0]).wait()
        pltpu.make_async_copy(buf_c.at[pp], ec_hbm.at[pl.ds(base, SUP)],
                              sem_w.at[pp, 1]).wait()


def _sc_gather(table, rows2, cols2):
    return pl.kernel(
        _gather_body,
        out_type=[jax.ShapeDtypeStruct((E_N, D), jnp.float32),
                  jax.ShapeDtypeStruct((E_N, D), jnp.float32)],
        mesh=_sc_mesh(),
        compiler_params=pltpu.CompilerParams(use_tc_tiling_on_sc=False),
        scratch_types=[
            pltpu.VMEM((NCHUNK, CH), jnp.int32),
            pltpu.VMEM((NCHUNK, CH), jnp.int32),
            pltpu.VMEM((2, NB * CH, D), jnp.float32),
            pltpu.VMEM((2, NB * CH, D), jnp.float32),
            pltpu.SemaphoreType.DMA,
            pltpu.SemaphoreType.DMA((2, 2)),
        ],
    )(table, rows2, cols2)


def _scatter_body(vals_hbm, rows_hbm, zeros_hbm, out_hbm,
                  idx_v, val_v, zbuf, obuf, acc_sh, sem_v):
    c = jax.lax.axis_index("c")
    s = jax.lax.axis_index("s")
    r0 = s * RPT
    w = c * NS + s
    base = w * EPW
    pltpu.sync_copy(rows_hbm.at[pl.ds(w * NCHUNK, NCHUNK)], idx_v)
    pltpu.sync_copy(zeros_hbm, zbuf)
    pltpu.sync_copy(zbuf, acc_sh.at[pl.ds(r0, RPT)])
    plsc.subcore_barrier()

    def outer(j, carry):
        i0 = j * NB
        ds = []
        for b in range(NB):
            off = base + (i0 + b) * CH
            ds.append(pltpu.async_copy(vals_hbm.at[pl.ds(off, CH)],
                                       val_v.at[b], sem_v))
        for b in range(NB):
            ds[b].wait()
            pltpu.sync_copy(val_v.at[b], acc_sh.at[idx_v.at[i0 + b]], add=True)
        return carry

    jax.lax.fori_loop(0, NOUT, outer, 0)
    plsc.subcore_barrier()
    pltpu.sync_copy(acc_sh.at[pl.ds(r0, RPT)], obuf)
    pltpu.sync_copy(obuf, out_hbm.at[c, pl.ds(r0, RPT)])


def _sc_scatter(vals, rows2, zeros_rpt):
    return pl.kernel(
        _scatter_body,
        out_type=jax.ShapeDtypeStruct((NC, L_N, D), jnp.float32),
        mesh=_sc_mesh(),
        compiler_params=pltpu.CompilerParams(use_tc_tiling_on_sc=False),
        scratch_types=[
            pltpu.VMEM((NCHUNK, CH), jnp.int32),
            pltpu.VMEM((NB, CH, D), jnp.float32),
            pltpu.VMEM((RPT, D), jnp.float32),
            pltpu.VMEM((RPT, D), jnp.float32),
            pltpu.VMEM_SHARED((L_N, D), jnp.float32),
            pltpu.SemaphoreType.DMA,
        ],
    )(vals, rows2, zeros_rpt)


def _embed_body(feat_ref, x_ref, w_ref, b_ref, out_ref):
    h = jnp.dot(feat_ref[...], w_ref[...], preferred_element_type=jnp.float32)
    h = h + b_ref[...]
    pad = jnp.zeros((L_N, D - HID - 3), jnp.float32)
    out_ref[...] = jnp.concatenate([h, x_ref[...], pad], axis=1)


def _tc_embed(in_feat, x0, w, b):
    return pl.pallas_call(
        _embed_body,
        out_shape=jax.ShapeDtypeStruct((L_N, D), jnp.float32),
    )(in_feat, x0, w, b)


BE4 = 3200
G = 4


def _bd4(mat, in_off=0, out_off=0):
    ki, ko = mat.shape
    kk, jj = np.meshgrid(np.arange(ki), np.arange(ko), indexing="ij")
    rows = np.concatenate([32 * g + in_off + kk.ravel() for g in range(G)])
    cols = np.concatenate([32 * g + out_off + jj.ravel() for g in range(G)])
    vals = jnp.tile(jnp.asarray(mat, jnp.float32).ravel(), G)
    return jnp.zeros((128, 128), jnp.float32).at[rows, cols].set(vals)


def _row4(vec, off=0):
    n = np.asarray(vec).shape[-1] if isinstance(vec, np.ndarray) else vec.shape[-1]
    cols = np.concatenate([32 * g + off + np.arange(n) for g in range(G)])
    vals = jnp.tile(jnp.asarray(vec, jnp.float32).ravel(), G)
    return jnp.zeros((1, 128), jnp.float32).at[0, cols].set(vals)


_RSUM_NP = np.zeros((128, 128), np.float32)
for _g in range(G):
    _RSUM_NP[32 * _g + HID:32 * _g + HID + 3, 32 * _g:32 * _g + HID + 3] = 1.0
_MASKX_NP = np.zeros((1, 128), np.float32)
_ONE29_NP = np.zeros((1, 128), np.float32)
for _g in range(G):
    _MASKX_NP[0, 32 * _g + HID:32 * _g + HID + 3] = 1.0
    _ONE29_NP[0, 32 * _g + HID + 3] = 1.0


def _edge_weights(p):
    return (
        _bd4(p["edge_w1"][:HID]),
        _bd4(p["edge_w1"][HID:2 * HID]),
        jnp.asarray(_RSUM_NP),
        _row4(p["edge_w1"][2 * HID]),
        _row4(p["edge_b1"]),
        _bd4(p["edge_w2"]),
        _row4(p["edge_b2"]),
        _bd4(jnp.broadcast_to(p["att_w"], (HID, HID))),
        _row4(jnp.broadcast_to(p["att_b"], (HID,))),
        _bd4(p["coord_w1"]),
        _row4(p["coord_b1"]),
        _bd4(jnp.broadcast_to(p["coord_w2"], (HID, 3)), out_off=HID),
        jnp.asarray(_MASKX_NP),
        jnp.asarray(_ONE29_NP),
    )


def _edge_body(er_ref, ec_ref, w1a, w1b, rsum, w1cr, b1r, w2, b2r, awd, abr,
               cw1, cb1r, cws, maskx, one29, out_ref):
    er = er_ref[...]
    ec = ec_ref[...]
    xd = (er - ec) * maskx[...]
    rb = jnp.dot(xd * xd, rsum[...], preferred_element_type=jnp.float32)
    cd = xd / (jnp.sqrt(rb) + 1e-8)
    m = (jnp.dot(er, w1a[...], preferred_element_type=jnp.float32)
         + jnp.dot(ec, w1b[...], preferred_element_type=jnp.float32)
         + rb * w1cr[...] + b1r[...])
    m = m * jax.nn.sigmoid(m)
    m = jnp.dot(m, w2[...], preferred_element_type=jnp.float32) + b2r[...]
    m = m * jax.nn.sigmoid(m)
    att = jax.nn.sigmoid(jnp.dot(m, awd[...], preferred_element_type=jnp.float32)
                         + abr[...])
    m = m * att
    tf = jnp.dot(m, cw1[...], preferred_element_type=jnp.float32) + cb1r[...]
    tf = tf * jax.nn.sigmoid(tf)
    scal = jnp.dot(tf, cws[...], preferred_element_type=jnp.float32)
    out_ref[...] = m + cd * scal + one29[...]


def _tc_edge(er4, ec4, wp):
    nblk = (E_N // G) // BE4
    full = [pl.BlockSpec((BE4, 128), lambda i: (i, 0)),
            pl.BlockSpec((BE4, 128), lambda i: (i, 0))] + [
        pl.BlockSpec(w.shape, lambda i, _n=w.ndim: (0,) * _n) for w in wp]
    return pl.pallas_call(
        _edge_body,
        grid=(nblk,),
        in_specs=full,
        out_specs=pl.BlockSpec((BE4, 128), lambda i: (i, 0)),
        out_shape=jax.ShapeDtypeStruct((E_N // G, 128), jnp.float32),
    )(er4, ec4, *wp)


BN = 1000


def _node_body(tab_ref, p0_ref, p1_ref, nw1a, nw1b, nb1, nw2, nb2, out_ref):
    tab = tab_ref[...]
    agg = p0_ref[...] + p1_ref[...]
    h = tab[:, :HID]
    x = tab[:, HID:HID + 3]
    m_agg = agg[:, :HID]
    tsum = agg[:, HID:HID + 3]
    cnt = agg[:, HID + 3:HID + 4]
    x_new = x + tsum / jnp.clip(cnt, 1.0, None)
    o = (jnp.dot(h, nw1a[...], preferred_element_type=jnp.float32)
         + jnp.dot(m_agg, nw1b[...], preferred_element_type=jnp.float32)
         + nb1[...])
    o = o * jax.nn.sigmoid(o)
    o = jnp.dot(o, nw2[...], preferred_element_type=jnp.float32) + nb2[...]
    h_new = h + o
    pad = jnp.zeros((BN, D - HID - 3), jnp.float32)
    out_ref[...] = jnp.concatenate([h_new, x_new, pad], axis=1)


def _tc_node(tab, p0, p1, wp):
    nspec = pl.BlockSpec((BN, D), lambda i: (i, 0))
    return pl.pallas_call(
        _node_body,
        grid=(L_N // BN,),
        in_specs=[nspec, nspec, nspec] + [
            pl.BlockSpec(w.shape, lambda i, _n=w.ndim: (0,) * _n) for w in wp],
        out_specs=nspec,
        out_shape=jax.ShapeDtypeStruct((L_N, D), jnp.float32),
    )(tab, p0, p1, *wp)


def _final_body(tab_ref, aux_ref, ew, eb, sc_ref, o_rot, o_pos, o_kld):
    tab = tab_ref[...]
    aux = aux_ref[...]
    h = tab[:, :HID]
    x = tab[:, HID:HID + 3]
    pred = jnp.dot(h, ew[...], preferred_element_type=jnp.float32) + eb[...]
    logits = pred[:, :KC]
    mx = jnp.max(logits, axis=1, keepdims=True)
    ex = jnp.exp(logits - mx)
    cden = ex / jnp.sum(ex, axis=1, keepdims=True)
    q = pred[:, KC:KC + 3]
    b_ = q[:, 0:1]
    c_ = q[:, 1:2]
    d_ = q[:, 2:3]
    sq = jnp.sqrt(1.0 + b_ * b_ + c_ * c_ + d_ * d_)
    a_ = 1.0 / sq
    b_ = b_ / sq
    c_ = c_ / sq
    d_ = d_ / sq
    U = [[a_ * a_ + b_ * b_ - c_ * c_ - d_ * d_, 2 * b_ * c_ - 2 * a_ * d_,
          2 * b_ * d_ + 2 * a_ * c_],
         [2 * b_ * c_ + 2 * a_ * d_, a_ * a_ - b_ * b_ + c_ * c_ - d_ * d_,
          2 * c_ * d_ - 2 * a_ * b_],
         [2 * b_ * d_ - 2 * a_ * c_, 2 * c_ * d_ + 2 * a_ * b_,
          a_ * a_ - b_ * b_ - c_ * c_ + d_ * d_]]
    Rn = [[aux[:, 3 * i + j:3 * i + j + 1] for j in range(3)] for i in range(3)]
    R0 = [[aux[:, 9 + 3 * i + j:9 + 3 * i + j + 1] for j in range(3)]
          for i in range(3)]
    Rp = [[Rn[i][0] * U[0][j] + Rn[i][1] * U[1][j] + Rn[i][2] * U[2][j]
           for j in range(3)] for i in range(3)]
    loss_rot = jnp.zeros((BN, 1), jnp.float32)
    for j in range(3):
        dot = Rp[0][j] * R0[0][j] + Rp[1][j] * R0[1][j] + Rp[2][j] * R0[2][j]
        na = jnp.sqrt(Rp[0][j] ** 2 + Rp[1][j] ** 2 + Rp[2][j] ** 2)
        nb = jnp.sqrt(R0[0][j] ** 2 + R0[1][j] ** 2 + R0[2][j] ** 2)
        cos = dot / jnp.maximum(na * nb, 1e-8)
        loss_rot = loss_rot + (1.0 - cos)
    pn = aux[:, 18:21]
    ep = aux[:, 21:24]
    dd = (x - pn) - ep
    pos = jnp.sum(dd * dd, axis=1, keepdims=True)
    alpha = sc_ref[0]
    abp = sc_ref[1]
    c0h = aux[:, 24:24 + KC]
    cth = aux[:, 24 + KC:24 + 2 * KC]
    t1 = alpha * cth + (1.0 - alpha) / KC
    th_true = t1 * (abp * c0h + (1.0 - abp) / KC)
    pt = th_true / (jnp.sum(th_true, axis=1, keepdims=True) + 1e-8)
    th_pred = t1 * (abp * cden + (1.0 - abp) / KC)
    pp = th_pred / (jnp.sum(th_pred, axis=1, keepdims=True) + 1e-8)
    lpp = jnp.log(pp + 1e-8)
    xlx = jnp.where(pt > 0.0, pt * jnp.log(jnp.maximum(pt, 1e-30)), 0.0)
    kld = jnp.sum(xlx - pt * lpp, axis=1, keepdims=True)

    @pl.when(pl.program_id(0) == 0)
    def _init():
        o_rot[...] = jnp.zeros((1, 1), jnp.float32)
        o_pos[...] = jnp.zeros((1, 1), jnp.float32)
        o_kld[...] = jnp.zeros((1, 1), jnp.float32)

    o_rot[...] += jnp.sum(loss_rot, keepdims=True)
    o_pos[...] += jnp.sum(pos, keepdims=True)
    o_kld[...] += jnp.sum(kld, keepdims=True)


def _tc_final(tab, aux, ew, eb, scal):
    ospec = pl.BlockSpec((1, 1), lambda i: (0, 0))
    return pl.pallas_call(
        _final_body,
        grid=(L_N // BN,),
        in_specs=[
            pl.BlockSpec((BN, D), lambda i: (i, 0)),
            pl.BlockSpec((BN, 64), lambda i: (i, 0)),
            pl.BlockSpec((HID, 23), lambda i: (0, 0)),
            pl.BlockSpec((1, 23), lambda i: (0, 0)),
            pl.BlockSpec(memory_space=pltpu.SMEM),
        ],
        out_specs=[ospec, ospec, ospec],
        out_shape=[jax.ShapeDtypeStruct((1, 1), jnp.float32)] * 3,
    )(tab, aux, ew, eb, scal)


def kernel(p_0, c_0, v_0, e, t, params):
    betas, alphas, alpha_bars = _schedule_k()
    nkey = jax.random.key(1234)
    k1, k2, k3, k4 = jax.random.split(nkey, 4)
    N, L = p_0.shape[0], p_0.shape[1]
    rows, cols = e[0], e[1]
    p_0n = p_0 / 10.0
    R_0 = _so3vec_to_rotation(v_0)
    ab_t = alpha_bars[t]
    c0 = jnp.sqrt(ab_t)[:, None, None]
    c1 = jnp.sqrt(1.0 - ab_t)[:, None, None]
    e_scaled = c1 * jax.random.normal(k1, v_0.shape, jnp.float32)
    E_scaled = _so3vec_to_rotation(e_scaled)
    R0_scaled = _so3vec_to_rotation(c0 * v_0)
    R_noisy = E_scaled @ R0_scaled
    v_noisy = _rotation_to_so3vec(R_noisy)
    eps_p = jax.random.normal(k2, p_0.shape, jnp.float32)
    p_noisy = c0 * p_0n + c1 * eps_p
    s_0 = jax.random.categorical(
        k3, jnp.log(jnp.clip(jax.lax.stop_gradient(c_0), 1e-8, None)), axis=-1)
    c0h = jax.nn.one_hot(s_0, KC, dtype=jnp.float32)
    c_noisy = ab_t[:, None, None] * c0h + (1.0 - ab_t[:, None, None]) / KC
    s_noisy = jax.random.categorical(
        k4, jnp.log(jnp.clip(jax.lax.stop_gradient(c_noisy), 1e-8, None)), axis=-1)
    beta = betas[t]
    t_embed = jnp.stack([beta, jnp.sin(beta), jnp.cos(beta)], axis=-1)
    t_embed = jnp.broadcast_to(t_embed[0][None, :], (L, 3))
    in_feat = jnp.concatenate([c_noisy[0], v_noisy[0], t_embed], axis=1)

    tab = _tc_embed(in_feat, p_noisy[0], params["emb_in_w"],
                    params["emb_in_b"].reshape(1, HID))
    zeros_rpt = jnp.zeros((RPT, D), jnp.float32)
    rows2 = rows.reshape(E_N // CH, CH)
    cols2 = cols.reshape(E_N // CH, CH)
    for p in params["layers"]:
        ewp = _edge_weights(p)
        nwp = (p["node_w1"][:HID], p["node_w1"][HID:2 * HID],
               p["node_b1"].reshape(1, HID), p["node_w2"],
               p["node_b2"].reshape(1, HID))
        er, ec = _sc_gather(tab, rows2, cols2)
        vals4 = _tc_edge(er.reshape(E_N // G, 128), ec.reshape(E_N // G, 128),
                         ewp)
        part = _sc_scatter(vals4.reshape(E_N, D), rows2, zeros_rpt)
        tab = _tc_node(tab, part[0], part[1], nwp)

    ct_h = jax.nn.one_hot(s_noisy, KC, dtype=jnp.float32)
    aux = jnp.concatenate([
        R_noisy[0].reshape(L, 9), R_0[0].reshape(L, 9),
        p_noisy[0], eps_p[0], c0h[0], ct_h[0]], axis=1)
    scal = jnp.stack([alphas[t][0],
                      alpha_bars[(t - 1) % alpha_bars.shape[0]][0]])
    s_rot, s_pos, s_kld = _tc_final(tab, aux, params["emb_out_w"],
                                    params["emb_out_b"].reshape(1, 23), scal)
    denom = float(N * L) + 1e-8
    return jnp.stack([s_rot[0, 0] / denom, s_pos[0, 0] / denom,
                      s_kld[0, 0] / denom])
